# baseline (device time: 29753 ns/iter reference)
import jax
import jax.numpy as jnp
from jax import lax
from jax.experimental import pallas as pl
from jax.experimental.pallas import tpu as pltpu

N_DEV = 4


def kernel(x, w_mat):
    m_total, k_per = x.shape
    _, n = w_mat.shape
    m_per = m_total // N_DEV

    def body(x_ref, w_ref, out_ref, comm_ref, send_sems, recv_sems):
        my = lax.axis_index("i")
        left = (my - 1) % N_DEV
        right = (my + 1) % N_DEV

        barrier_sem = pltpu.get_barrier_semaphore()
        for nbr in [left, right]:
            pl.semaphore_signal(
                barrier_sem, inc=1,
                device_id=(nbr,), device_id_type=pl.DeviceIdType.MESH,
            )
        pl.semaphore_wait(barrier_sem, 2)

        def partial(c):
            return jnp.dot(
                x_ref[pl.ds(c * m_per, m_per), :],
                w_ref[:, :],
                preferred_element_type=jnp.float32,
            )

        comm_ref[0, :, :] = partial((my - 1) % N_DEV).astype(jnp.bfloat16)

        for h in range(N_DEV - 1):
            rdma = pltpu.make_async_remote_copy(
                src_ref=comm_ref.at[h],
                dst_ref=comm_ref.at[h + 1],
                send_sem=send_sems.at[h],
                recv_sem=recv_sems.at[h],
                device_id=(right,),
                device_id_type=pl.DeviceIdType.MESH,
            )
            rdma.start()
            rdma.wait()

            c = (my - 2 - h) % N_DEV
            acc = comm_ref[h + 1, :, :].astype(jnp.float32) + partial(c)
            if h < N_DEV - 2:
                comm_ref[h + 1, :, :] = acc.astype(jnp.bfloat16)
            else:
                out_ref[:, :] = jnp.maximum(acc, 0.0)

    return pl.pallas_call(
        body,
        out_shape=jax.ShapeDtypeStruct((m_per, n), jnp.float32),
        in_specs=[
            pl.BlockSpec(memory_space=pltpu.VMEM),
            pl.BlockSpec(memory_space=pltpu.VMEM),
        ],
        out_specs=pl.BlockSpec(memory_space=pltpu.VMEM),
        scratch_shapes=[
            pltpu.VMEM((N_DEV, m_per, n), jnp.bfloat16),
            pltpu.SemaphoreType.DMA((N_DEV - 1,)),
            pltpu.SemaphoreType.DMA((N_DEV - 1,)),
        ],
        compiler_params=pltpu.CompilerParams(collective_id=0),
    )(x, w_mat)


# device time: 17391 ns/iter; 1.7108x vs baseline; 1.7108x over previous
import jax
import jax.numpy as jnp
from jax import lax
from jax.experimental import pallas as pl
from jax.experimental.pallas import tpu as pltpu

N_DEV = 4

X_FWD, X_FIN, X_P2 = 0, 1, 2
Y_FWD, Y_FIN, Y_P2 = 3, 4, 5


def kernel(x, w_mat):
    m_total, k_per = x.shape
    _, n = w_mat.shape
    m_per = m_total // N_DEV
    h = n // 2

    def body(x_ref, w_ref, out_ref, outs, ins, send_sems, recv_sems):
        my = lax.axis_index("i")
        xp = 3 - my
        yp = my ^ 1
        cdiag = (my + 2) % N_DEV

        barrier_sem = pltpu.get_barrier_semaphore()
        for nbr in [xp, yp]:
            pl.semaphore_signal(
                barrier_sem, inc=1,
                device_id=(nbr,), device_id_type=pl.DeviceIdType.MESH,
            )
        pl.semaphore_wait(barrier_sem, 2)

        def rows(c):
            return x_ref[pl.ds(c * m_per, m_per), :]

        def mk(k, target):
            return pltpu.make_async_remote_copy(
                src_ref=outs.at[k],
                dst_ref=ins.at[k],
                send_sem=send_sems.at[k],
                recv_sem=recv_sems.at[k],
                device_id=(target,),
                device_id_type=pl.DeviceIdType.MESH,
            )

        f32 = jnp.float32
        bf16 = jnp.bfloat16

        pdiag = jnp.dot(rows(cdiag), w_ref[:, :], preferred_element_type=f32)
        outs[X_FWD, :, :] = pdiag[:, :h].astype(bf16)
        mk(X_FWD, xp).start()
        outs[Y_FWD, :, :] = pdiag[:, h:].astype(bf16)
        mk(Y_FWD, yp).start()

        outs[X_FIN, :, :] = jnp.dot(
            rows(xp), w_ref[:, :h], preferred_element_type=f32
        ).astype(bf16)
        mk(X_FIN, xp).start()
        outs[Y_FIN, :, :] = jnp.dot(
            rows(yp), w_ref[:, h:], preferred_element_type=f32
        ).astype(bf16)
        mk(Y_FIN, yp).start()

        my_h1_yp = jnp.dot(rows(yp), w_ref[:, :h], preferred_element_type=f32)
        my_h2_xp = jnp.dot(rows(xp), w_ref[:, h:], preferred_element_type=f32)

        mk(X_FWD, my).wait_recv()
        outs[Y_P2, :, :] = (ins[X_FWD, :, :].astype(f32) + my_h1_yp).astype(bf16)
        mk(Y_P2, yp).start()
        mk(Y_FWD, my).wait_recv()
        outs[X_P2, :, :] = (ins[Y_FWD, :, :].astype(f32) + my_h2_xp).astype(bf16)
        mk(X_P2, xp).start()

        pmine = jnp.dot(rows(my), w_ref[:, :], preferred_element_type=f32)

        mk(X_FIN, my).wait_recv()
        mk(Y_P2, my).wait_recv()
        out_ref[:, :h] = jnp.maximum(
            pmine[:, :h]
            + ins[X_FIN, :, :].astype(f32)
            + ins[Y_P2, :, :].astype(f32),
            0.0,
        )
        mk(Y_FIN, my).wait_recv()
        mk(X_P2, my).wait_recv()
        out_ref[:, h:] = jnp.maximum(
            pmine[:, h:]
            + ins[Y_FIN, :, :].astype(f32)
            + ins[X_P2, :, :].astype(f32),
            0.0,
        )

        for k in range(6):
            mk(k, my).wait_send()

    return pl.pallas_call(
        body,
        out_shape=jax.ShapeDtypeStruct((m_per, n), jnp.float32),
        in_specs=[
            pl.BlockSpec(memory_space=pltpu.VMEM),
            pl.BlockSpec(memory_space=pltpu.VMEM),
        ],
        out_specs=pl.BlockSpec(memory_space=pltpu.VMEM),
        scratch_shapes=[
            pltpu.VMEM((6, m_per, h), jnp.bfloat16),
            pltpu.VMEM((6, m_per, h), jnp.bfloat16),
            pltpu.SemaphoreType.DMA((6,)),
            pltpu.SemaphoreType.DMA((6,)),
        ],
        compiler_params=pltpu.CompilerParams(collective_id=0),
    )(x, w_mat)


# device time: 17038 ns/iter; 1.7463x vs baseline; 1.0207x over previous
import jax
import jax.numpy as jnp
from jax import lax
from jax.experimental import pallas as pl
from jax.experimental.pallas import tpu as pltpu

N_DEV = 4
NSLOT = 12

XF0, XF1, XN0, XN1, XP2, XP3 = 0, 1, 2, 3, 4, 5
YF2, YF3, YN2, YN3, YP0, YP1 = 6, 7, 8, 9, 10, 11


def kernel(x, w_mat):
    m_total, k_per = x.shape
    _, n = w_mat.shape
    m_per = m_total // N_DEV
    q = n // 4

    def body(x_ref, w_ref, out_ref, outs, ins, send_sems, recv_sems):
        my = lax.axis_index("i")
        xp = 3 - my
        yp = my ^ 1
        cdiag = (my + 2) % N_DEV

        barrier_sem = pltpu.get_barrier_semaphore()
        for nbr in [xp, yp]:
            pl.semaphore_signal(
                barrier_sem, inc=1,
                device_id=(nbr,), device_id_type=pl.DeviceIdType.MESH,
            )

        f32 = jnp.float32
        bf16 = jnp.bfloat16

        def part(c, qi):
            return jnp.dot(
                x_ref[pl.ds(c * m_per, m_per), :],
                w_ref[:, qi * q:(qi + 1) * q],
                preferred_element_type=f32,
            )

        def mk(k, target):
            return pltpu.make_async_remote_copy(
                src_ref=outs.at[k],
                dst_ref=ins.at[k],
                send_sem=send_sems.at[k],
                recv_sem=recv_sems.at[k],
                device_id=(target,),
                device_id_type=pl.DeviceIdType.MESH,
            )

        def gemm_send(slot, c, qi, target):
            outs[slot, :, :] = part(c, qi).astype(bf16)
            mk(slot, target).start()

        outs[XF0, :, :] = part(cdiag, 0).astype(bf16)
        outs[YF2, :, :] = part(cdiag, 2).astype(bf16)
        pl.semaphore_wait(barrier_sem, 2)
        mk(XF0, xp).start()
        mk(YF2, yp).start()
        gemm_send(XF1, cdiag, 1, xp)
        gemm_send(YF3, cdiag, 3, yp)
        gemm_send(XN0, xp, 0, xp)
        gemm_send(YN2, yp, 2, yp)
        gemm_send(XN1, xp, 1, xp)
        gemm_send(YN3, yp, 3, yp)

        my_yp0 = part(yp, 0)
        my_yp1 = part(yp, 1)
        my_xp2 = part(xp, 2)
        my_xp3 = part(xp, 3)

        mk(XF0, my).wait_recv()
        outs[YP0, :, :] = (ins[XF0, :, :].astype(f32) + my_yp0).astype(bf16)
        mk(YP0, yp).start()
        mk(YF2, my).wait_recv()
        outs[XP2, :, :] = (ins[YF2, :, :].astype(f32) + my_xp2).astype(bf16)
        mk(XP2, xp).start()
        mk(XF1, my).wait_recv()
        outs[YP1, :, :] = (ins[XF1, :, :].astype(f32) + my_yp1).astype(bf16)
        mk(YP1, yp).start()
        mk(YF3, my).wait_recv()
        outs[XP3, :, :] = (ins[YF3, :, :].astype(f32) + my_xp3).astype(bf16)
        mk(XP3, xp).start()

        pre = [None] * 4
        for qi, slot in ((0, XN0), (2, YN2), (1, XN1), (3, YN3)):
            pmine = part(my, qi)
            mk(slot, my).wait_recv()
            pre[qi] = pmine + ins[slot, :, :].astype(f32)

        for qi, slot in ((0, YP0), (2, XP2), (1, YP1), (3, XP3)):
            mk(slot, my).wait_recv()
            out_ref[:, qi * q:(qi + 1) * q] = jnp.maximum(
                pre[qi] + ins[slot, :, :].astype(f32), 0.0
            )

        for k in range(NSLOT):
            mk(k, my).wait_send()

    return pl.pallas_call(
        body,
        out_shape=jax.ShapeDtypeStruct((m_per, n), jnp.float32),
        in_specs=[
            pl.BlockSpec(memory_space=pltpu.VMEM),
            pl.BlockSpec(memory_space=pltpu.VMEM),
        ],
        out_specs=pl.BlockSpec(memory_space=pltpu.VMEM),
        scratch_shapes=[
            pltpu.VMEM((NSLOT, m_per, q), jnp.bfloat16),
            pltpu.VMEM((NSLOT, m_per, q), jnp.bfloat16),
            pltpu.SemaphoreType.DMA((NSLOT,)),
            pltpu.SemaphoreType.DMA((NSLOT,)),
        ],
        compiler_params=pltpu.CompilerParams(collective_id=0),
    )(x, w_mat)
